# baseline (device time: 50004 ns/iter reference)
import jax
import jax.numpy as jnp
from jax import lax
from jax.experimental import pallas as pl
from jax.experimental.pallas import tpu as pltpu

N_DEV = 16
N_PLANE = 4
N_Z = 4
M = 2048
M_CHUNK = M // N_DEV

_SCALE = 160.0 / 127.0


def kernel(A, B):
    m, k_loc = A.shape
    _, n = B.shape

    def body(
        a_ref, b_ref, out_ref,
        qbuf, recv1, send2, recv2,
        send_sems1, recv_sems1, send_sems2, recv_sems2,
    ):
        me = lax.axis_index("i")
        z = me // N_PLANE
        c = me % N_PLANE

        barrier_sem = pltpu.get_barrier_semaphore()
        partners = [z * N_PLANE + (c + kp) % N_PLANE for kp in range(1, N_PLANE)]
        partners += [((z + kz) % N_Z) * N_PLANE + c for kz in range(1, N_Z)]
        for p in partners:
            pl.semaphore_signal(
                barrier_sem, inc=1,
                device_id=(p,), device_id_type=pl.DeviceIdType.MESH,
            )
        pl.semaphore_wait(barrier_sem, len(partners))

        def quant(x):
            return jnp.clip(
                jnp.round(x * (1.0 / _SCALE)), -127.0, 127.0
            ).astype(jnp.int8)

        sends = []
        rows_per_round = N_PLANE * M_CHUNK
        for zi in range(N_Z):
            zz = (z + 1 + zi) % N_Z
            r = jnp.dot(
                a_ref[pl.ds(zz * rows_per_round, rows_per_round), :],
                b_ref[:, :],
                preferred_element_type=jnp.float32,
            )
            qbuf[zi] = quant(r)
            recv1[zz, c] = qbuf[zi, pl.ds(c * M_CHUNK, M_CHUNK)]
            for kp in range(N_PLANE - 1):
                cp = (c + 1 + kp) % N_PLANE
                rdma = pltpu.make_async_remote_copy(
                    src_ref=qbuf.at[zi, pl.ds(cp * M_CHUNK, M_CHUNK)],
                    dst_ref=recv1.at[zz, c],
                    send_sem=send_sems1.at[kp, zi],
                    recv_sem=recv_sems1.at[zz, c],
                    device_id=(z * N_PLANE + cp,),
                    device_id_type=pl.DeviceIdType.MESH,
                )
                rdma.start()
                sends.append(rdma)

        for zi in range(N_Z):
            zz = (z + 1 + zi) % N_Z
            for kp in range(N_PLANE - 1):
                cp = (c + 1 + kp) % N_PLANE
                recv = pltpu.make_async_remote_copy(
                    src_ref=qbuf.at[zi, pl.ds(cp * M_CHUNK, M_CHUNK)],
                    dst_ref=recv1.at[zz, cp],
                    send_sem=send_sems1.at[kp, zi],
                    recv_sem=recv_sems1.at[zz, cp],
                    device_id=(z * N_PLANE + cp,),
                    device_id_type=pl.DeviceIdType.MESH,
                )
                recv.wait_recv()
            s = jnp.sum(recv1[zz].astype(jnp.float32), axis=0) * _SCALE
            if zi < N_Z - 1:
                send2[zi] = s.astype(jnp.bfloat16)
                rdma = pltpu.make_async_remote_copy(
                    src_ref=send2.at[zi],
                    dst_ref=recv2.at[z],
                    send_sem=send_sems2.at[zi],
                    recv_sem=recv_sems2.at[z],
                    device_id=(zz * N_PLANE + c,),
                    device_id_type=pl.DeviceIdType.MESH,
                )
                rdma.start()
                sends.append(rdma)
            else:
                recv2[z] = s.astype(jnp.bfloat16)

        for kz in range(1, N_Z):
            zp = (z + kz) % N_Z
            recv = pltpu.make_async_remote_copy(
                src_ref=send2.at[0],
                dst_ref=recv2.at[zp],
                send_sem=send_sems2.at[0],
                recv_sem=recv_sems2.at[zp],
                device_id=(zp * N_PLANE + c,),
                device_id_type=pl.DeviceIdType.MESH,
            )
            recv.wait_recv()

        for rdma in sends:
            rdma.wait_send()

        out_ref[:, :] = jnp.sum(recv2[:, :, :].astype(jnp.float32), axis=0)

    return pl.pallas_call(
        body,
        out_shape=jax.ShapeDtypeStruct((M_CHUNK, n), jnp.float32),
        in_specs=[
            pl.BlockSpec(memory_space=pltpu.VMEM),
            pl.BlockSpec(memory_space=pltpu.VMEM),
        ],
        out_specs=pl.BlockSpec(memory_space=pltpu.VMEM),
        scratch_shapes=[
            pltpu.VMEM((N_Z, N_PLANE * M_CHUNK, n), jnp.int8),
            pltpu.VMEM((N_Z, N_PLANE, M_CHUNK, n), jnp.int8),
            pltpu.VMEM((N_Z - 1, M_CHUNK, n), jnp.bfloat16),
            pltpu.VMEM((N_Z, M_CHUNK, n), jnp.bfloat16),
            pltpu.SemaphoreType.DMA((N_PLANE - 1, N_Z)),
            pltpu.SemaphoreType.DMA((N_Z, N_PLANE)),
            pltpu.SemaphoreType.DMA((N_Z - 1,)),
            pltpu.SemaphoreType.DMA((N_Z,)),
        ],
        compiler_params=pltpu.CompilerParams(
            collective_id=0, vmem_limit_bytes=100 * 1024 * 1024
        ),
    )(A, B)


# device time: 43532 ns/iter; 1.1487x vs baseline; 1.1487x over previous
import jax
import jax.numpy as jnp
from jax import lax
from jax.experimental import pallas as pl
from jax.experimental.pallas import tpu as pltpu

N_DEV = 16
N_PLANE = 4
N_Z = 4
M = 2048
M_CHUNK = M // N_DEV

_SCALE = 160.0 / 127.0
_SCALE2 = 320.0 / 127.0


def kernel(A, B):
    m, k_loc = A.shape
    _, n = B.shape

    def body(
        a_ref, b_ref, out_ref,
        qbuf, recv1, send2, recv2,
        send_sems1, recv_sems1, send_sems2, recv_sems2,
    ):
        me = lax.axis_index("i")
        z = me // N_PLANE
        c = me % N_PLANE

        barrier_sem = pltpu.get_barrier_semaphore()
        partners = [z * N_PLANE + (c + kp) % N_PLANE for kp in range(1, N_PLANE)]
        partners += [((z + kz) % N_Z) * N_PLANE + c for kz in range(1, N_Z)]
        for p in partners:
            pl.semaphore_signal(
                barrier_sem, inc=1,
                device_id=(p,), device_id_type=pl.DeviceIdType.MESH,
            )
        pl.semaphore_wait(barrier_sem, len(partners))

        def quant(x, scale=_SCALE):
            return jnp.clip(
                jnp.round(x * (1.0 / scale)), -127.0, 127.0
            ).astype(jnp.int8)

        sends = []
        rows_per_round = N_PLANE * M_CHUNK
        for zi in range(N_Z):
            zz = (z + 1 + zi) % N_Z
            r = jnp.dot(
                a_ref[pl.ds(zz * rows_per_round, rows_per_round), :],
                b_ref[:, :],
                preferred_element_type=jnp.float32,
            )
            qbuf[zi] = quant(r)
            recv1[zz, c] = qbuf[zi, pl.ds(c * M_CHUNK, M_CHUNK)]
            for kp in range(N_PLANE - 1):
                cp = (c + 1 + kp) % N_PLANE
                rdma = pltpu.make_async_remote_copy(
                    src_ref=qbuf.at[zi, pl.ds(cp * M_CHUNK, M_CHUNK)],
                    dst_ref=recv1.at[zz, c],
                    send_sem=send_sems1.at[kp, zi],
                    recv_sem=recv_sems1.at[zz, c],
                    device_id=(z * N_PLANE + cp,),
                    device_id_type=pl.DeviceIdType.MESH,
                )
                rdma.start()
                sends.append(rdma)

        for zi in range(N_Z):
            zz = (z + 1 + zi) % N_Z
            for kp in range(N_PLANE - 1):
                cp = (c + 1 + kp) % N_PLANE
                recv = pltpu.make_async_remote_copy(
                    src_ref=qbuf.at[zi, pl.ds(cp * M_CHUNK, M_CHUNK)],
                    dst_ref=recv1.at[zz, cp],
                    send_sem=send_sems1.at[kp, zi],
                    recv_sem=recv_sems1.at[zz, cp],
                    device_id=(z * N_PLANE + cp,),
                    device_id_type=pl.DeviceIdType.MESH,
                )
                recv.wait_recv()
            s = jnp.sum(recv1[zz].astype(jnp.float32), axis=0) * _SCALE
            if zi < N_Z - 1:
                send2[zi] = quant(s, _SCALE2)
                rdma = pltpu.make_async_remote_copy(
                    src_ref=send2.at[zi],
                    dst_ref=recv2.at[z],
                    send_sem=send_sems2.at[zi],
                    recv_sem=recv_sems2.at[z],
                    device_id=(zz * N_PLANE + c,),
                    device_id_type=pl.DeviceIdType.MESH,
                )
                rdma.start()
                sends.append(rdma)
            else:
                s_own = s

        for kz in range(1, N_Z):
            zp = (z + kz) % N_Z
            recv = pltpu.make_async_remote_copy(
                src_ref=send2.at[0],
                dst_ref=recv2.at[zp],
                send_sem=send_sems2.at[0],
                recv_sem=recv_sems2.at[zp],
                device_id=(zp * N_PLANE + c,),
                device_id_type=pl.DeviceIdType.MESH,
            )
            recv.wait_recv()

        for rdma in sends:
            rdma.wait_send()

        acc = s_own
        for kz in range(1, N_Z):
            zp = (z + kz) % N_Z
            acc = acc + recv2[zp].astype(jnp.float32) * _SCALE2
        out_ref[:, :] = acc

    return pl.pallas_call(
        body,
        out_shape=jax.ShapeDtypeStruct((M_CHUNK, n), jnp.float32),
        in_specs=[
            pl.BlockSpec(memory_space=pltpu.VMEM),
            pl.BlockSpec(memory_space=pltpu.VMEM),
        ],
        out_specs=pl.BlockSpec(memory_space=pltpu.VMEM),
        scratch_shapes=[
            pltpu.VMEM((N_Z, N_PLANE * M_CHUNK, n), jnp.int8),
            pltpu.VMEM((N_Z, N_PLANE, M_CHUNK, n), jnp.int8),
            pltpu.VMEM((N_Z - 1, M_CHUNK, n), jnp.int8),
            pltpu.VMEM((N_Z, M_CHUNK, n), jnp.int8),
            pltpu.SemaphoreType.DMA((N_PLANE - 1, N_Z)),
            pltpu.SemaphoreType.DMA((N_Z, N_PLANE)),
            pltpu.SemaphoreType.DMA((N_Z - 1,)),
            pltpu.SemaphoreType.DMA((N_Z,)),
        ],
        compiler_params=pltpu.CompilerParams(
            collective_id=0, vmem_limit_bytes=100 * 1024 * 1024
        ),
    )(A, B)


# device time: 40581 ns/iter; 1.2322x vs baseline; 1.0727x over previous
import jax
import jax.numpy as jnp
from jax import lax
from jax.experimental import pallas as pl
from jax.experimental.pallas import tpu as pltpu

N_DEV = 16
N_PLANE = 4
N_Z = 4
M = 2048
M_CHUNK = M // N_DEV

_SCALE = 160.0 / 127.0
_SCALE2 = 320.0 / 127.0


def kernel(A, B):
    m, k_loc = A.shape
    _, n = B.shape

    def body(
        a_ref, b_ref, out_ref,
        qbuf, recv1, send2, recv2,
        send_sems1, recv_sems1, send_sems2, recv_sems2,
    ):
        me = lax.axis_index("i")
        z = me // N_PLANE
        c = me % N_PLANE

        barrier_sem = pltpu.get_barrier_semaphore()
        partners = [z * N_PLANE + (c + kp) % N_PLANE for kp in range(1, N_PLANE)]
        partners += [((z + kz) % N_Z) * N_PLANE + c for kz in range(1, N_Z)]
        for p in partners:
            pl.semaphore_signal(
                barrier_sem, inc=1,
                device_id=(p,), device_id_type=pl.DeviceIdType.MESH,
            )

        def quant(x, scale=_SCALE):
            return jnp.clip(
                jnp.round(x * (1.0 / scale)), -127.0, 127.0
            ).astype(jnp.int8)

        sends = []
        rows_per_round = N_PLANE * M_CHUNK
        for zi in range(N_Z):
            zz = (z + 1 + zi) % N_Z
            r = jnp.dot(
                a_ref[pl.ds(zz * rows_per_round, rows_per_round), :],
                b_ref[:, :],
                preferred_element_type=jnp.float32,
            )
            qbuf[zi] = quant(r)
            recv1[zz, c] = qbuf[zi, pl.ds(c * M_CHUNK, M_CHUNK)]
            if zi == 0:
                pl.semaphore_wait(barrier_sem, len(partners))
            for kp in range(N_PLANE - 1):
                cp = (c + 1 + kp) % N_PLANE
                rdma = pltpu.make_async_remote_copy(
                    src_ref=qbuf.at[zi, pl.ds(cp * M_CHUNK, M_CHUNK)],
                    dst_ref=recv1.at[zz, c],
                    send_sem=send_sems1.at[kp, zi],
                    recv_sem=recv_sems1.at[zz, c],
                    device_id=(z * N_PLANE + cp,),
                    device_id_type=pl.DeviceIdType.MESH,
                )
                rdma.start()
                sends.append(rdma)

        for zi in range(N_Z):
            zz = (z + 1 + zi) % N_Z
            for kp in range(N_PLANE - 1):
                cp = (c + 1 + kp) % N_PLANE
                recv = pltpu.make_async_remote_copy(
                    src_ref=qbuf.at[zi, pl.ds(cp * M_CHUNK, M_CHUNK)],
                    dst_ref=recv1.at[zz, cp],
                    send_sem=send_sems1.at[kp, zi],
                    recv_sem=recv_sems1.at[zz, cp],
                    device_id=(z * N_PLANE + cp,),
                    device_id_type=pl.DeviceIdType.MESH,
                )
                recv.wait_recv()
            s = jnp.sum(recv1[zz].astype(jnp.float32), axis=0) * _SCALE
            if zi < N_Z - 1:
                send2[zi] = quant(s, _SCALE2)
                rdma = pltpu.make_async_remote_copy(
                    src_ref=send2.at[zi],
                    dst_ref=recv2.at[z],
                    send_sem=send_sems2.at[zi],
                    recv_sem=recv_sems2.at[z],
                    device_id=(zz * N_PLANE + c,),
                    device_id_type=pl.DeviceIdType.MESH,
                )
                rdma.start()
                sends.append(rdma)
            else:
                s_own = s

        for kz in range(1, N_Z):
            zp = (z + kz) % N_Z
            recv = pltpu.make_async_remote_copy(
                src_ref=send2.at[0],
                dst_ref=recv2.at[zp],
                send_sem=send_sems2.at[0],
                recv_sem=recv_sems2.at[zp],
                device_id=(zp * N_PLANE + c,),
                device_id_type=pl.DeviceIdType.MESH,
            )
            recv.wait_recv()

        for rdma in sends:
            rdma.wait_send()

        acc = s_own
        for kz in range(1, N_Z):
            zp = (z + kz) % N_Z
            acc = acc + recv2[zp].astype(jnp.float32) * _SCALE2
        out_ref[:, :] = acc

    return pl.pallas_call(
        body,
        out_shape=jax.ShapeDtypeStruct((M_CHUNK, n), jnp.float32),
        in_specs=[
            pl.BlockSpec(memory_space=pltpu.VMEM),
            pl.BlockSpec(memory_space=pltpu.VMEM),
        ],
        out_specs=pl.BlockSpec(memory_space=pltpu.VMEM),
        scratch_shapes=[
            pltpu.VMEM((N_Z, N_PLANE * M_CHUNK, n), jnp.int8),
            pltpu.VMEM((N_Z, N_PLANE, M_CHUNK, n), jnp.int8),
            pltpu.VMEM((N_Z - 1, M_CHUNK, n), jnp.int8),
            pltpu.VMEM((N_Z, M_CHUNK, n), jnp.int8),
            pltpu.SemaphoreType.DMA((N_PLANE - 1, N_Z)),
            pltpu.SemaphoreType.DMA((N_Z, N_PLANE)),
            pltpu.SemaphoreType.DMA((N_Z - 1,)),
            pltpu.SemaphoreType.DMA((N_Z,)),
        ],
        compiler_params=pltpu.CompilerParams(
            collective_id=0, vmem_limit_bytes=100 * 1024 * 1024
        ),
    )(A, B)
